# baseline (speedup 1.0000x reference)
import functools

import jax
import jax.numpy as jnp
from jax import lax
from jax.experimental import pallas as pl
from jax.experimental.pallas import tpu as pltpu
from jax.experimental.pallas import tpu_sc as plsc

_D = 768
_LANES = 16
_NC = 2
_NS = 16
_NW = _NC * _NS
_C = 16
_NBUF = 3
_BB = 16


def _gather_body(tok_hbm, tidx_hbm, out_hbm, tidx_v,
                 t0, t1, t2, g0, g1, g2, s0, s1, s2, *, per_w, ll, lp):
    wid = lax.axis_index("s") * _NC + lax.axis_index("c")
    base = wid * per_w
    nch = per_w // _C
    lane = lax.iota(jnp.int32, _LANES)
    tbuf = (t0, t1, t2)
    gsem = (g0, g1, g2)
    ssem = (s0, s1, s2)

    pltpu.sync_copy(tidx_hbm.at[pl.ds(base, per_w)], tidx_v)

    def issue_tok(ci, s):
        pltpu.async_copy(tok_hbm.at[tidx_v.at[pl.ds(ci * _C, _C)]],
                         tbuf[s], gsem[s])

    def wait_tok(ci, s):
        pltpu.make_async_copy(tok_hbm.at[tidx_v.at[pl.ds(ci * _C, _C)]],
                              tbuf[s], gsem[s]).wait()

    def out_rows(ci):
        r = base + ci * _C + lane
        bv = r // ll
        return bv * lp + (r - bv * ll)

    def issue_scatter(ci, s):
        pltpu.async_copy(tbuf[s], out_hbm.at[out_rows(ci)], ssem[s])

    def wait_scatter(ci, s):
        pltpu.make_async_copy(tbuf[s], out_hbm.at[out_rows(ci)],
                              ssem[s]).wait()

    def step(ci, s):
        if ci >= 1:
            wait_scatter(ci - 1, (ci + 2) % _NBUF)
        if ci + 2 < nch:
            issue_tok(ci + 2, (ci + 2) % _NBUF)
        wait_tok(ci, s)
        issue_scatter(ci, s)

    issue_tok(0, 0)
    issue_tok(1, 1)
    step(0, 0)
    step(1, 1)

    def outer(g, _):
        for sp in range(_NBUF):
            ci = 2 + g * _NBUF + sp
            s = (2 + sp) % _NBUF
            wait_scatter(ci - 1, (s + 2) % _NBUF)
            issue_tok(ci + 2, (s + 2) % _NBUF)
            wait_tok(ci, s)
            issue_scatter(ci, s)
        return ()

    lax.fori_loop(0, (nch - 4) // _NBUF, outer, (), unroll=False)

    step(nch - 2, (nch - 2) % _NBUF)
    step(nch - 1, (nch - 1) % _NBUF)
    wait_scatter(nch - 1, (nch - 1) % _NBUF)


def _add_body(pos_ref, ptab_ref, tok_ref, out_ref, *, lp):
    ll = ptab_ref.shape[0]
    bb = out_ref.shape[0]
    posf = pos_ref[...]
    iota = lax.broadcasted_iota(jnp.int32, (1, ll), 1)
    onehot = (posf == iota).astype(jnp.float32)
    pe = lax.dot_general(onehot, ptab_ref[...], (((1,), (0,)), ((), ())),
                         precision=lax.Precision.HIGHEST,
                         preferred_element_type=jnp.float32)
    sm = (tok_ref[...] + pe).reshape(bb, lp, _D)
    out_ref[...] = sm[:, :ll, :]


def kernel(token_table, pos_table, tokens, positions):
    b, l = tokens.shape
    lp = (l + 7) // 8 * 8
    bt = b * l
    per_w = bt // _NW
    assert per_w % _C == 0 and (per_w // _C - 4) % _NBUF == 0

    tidx = tokens.reshape(bt).astype(jnp.int32)

    mesh = plsc.VectorSubcoreMesh(core_axis_name="c", subcore_axis_name="s")
    gather = pl.kernel(
        functools.partial(_gather_body, per_w=per_w, ll=l, lp=lp),
        mesh=mesh,
        compiler_params=pltpu.CompilerParams(needs_layout_passes=False),
        out_type=jax.ShapeDtypeStruct((b * lp, _D), jnp.float32),
        scratch_types=[
            pltpu.VMEM((per_w,), jnp.int32),
        ] + [pltpu.VMEM((_C, _D), jnp.float32)] * _NBUF
          + [pltpu.SemaphoreType.DMA] * (2 * _NBUF),
    )
    tok_rows = gather(token_table, tidx)

    pos_pad = jnp.pad(positions.astype(jnp.int32),
                      ((0, 0), (0, lp - l))).reshape(b * lp, 1)

    add = pl.pallas_call(
        functools.partial(_add_body, lp=lp),
        grid=(b // _BB,),
        in_specs=[
            pl.BlockSpec((_BB * lp, 1), lambda i: (i, 0)),
            pl.BlockSpec((l, _D), lambda i: (0, 0)),
            pl.BlockSpec((_BB * lp, _D), lambda i: (i, 0)),
        ],
        out_specs=pl.BlockSpec((_BB, l, _D), lambda i: (i, 0, 0)),
        out_shape=jax.ShapeDtypeStruct((b, l, _D), jnp.float32),
    )
    return add(pos_pad, pos_table, tok_rows)


# speedup vs baseline: 1.1241x; 1.1241x over previous
import functools

import jax
import jax.numpy as jnp
from jax import lax
from jax.experimental import pallas as pl
from jax.experimental.pallas import tpu as pltpu
from jax.experimental.pallas import tpu_sc as plsc

_D = 768
_LANES = 16
_NC = 2
_NS = 16
_NW = _NC * _NS
_C = 16
_NBUF = 3
_BB = 32


def _gather_body(tok_hbm, tidx_hbm, out_hbm, tidx_v,
                 t0, t1, t2, g0, g1, g2, s0, s1, s2, *, per_w, ll, lp):
    wid = lax.axis_index("s") * _NC + lax.axis_index("c")
    base = wid * per_w
    nch = per_w // _C
    lane = lax.iota(jnp.int32, _LANES)
    tbuf = (t0, t1, t2)
    gsem = (g0, g1, g2)
    ssem = (s0, s1, s2)

    pltpu.sync_copy(tidx_hbm.at[pl.ds(base, per_w)], tidx_v)

    def issue_tok(ci, s):
        pltpu.async_copy(tok_hbm.at[tidx_v.at[pl.ds(ci * _C, _C)]],
                         tbuf[s], gsem[s])

    def wait_tok(ci, s):
        pltpu.make_async_copy(tok_hbm.at[tidx_v.at[pl.ds(ci * _C, _C)]],
                              tbuf[s], gsem[s]).wait()

    def out_rows(ci):
        r = base + ci * _C + lane
        bv = r // ll
        return bv * lp + (r - bv * ll)

    def issue_scatter(ci, s):
        pltpu.async_copy(tbuf[s], out_hbm.at[out_rows(ci)], ssem[s])

    def wait_scatter(ci, s):
        pltpu.make_async_copy(tbuf[s], out_hbm.at[out_rows(ci)],
                              ssem[s]).wait()

    def step(ci, s):
        if ci >= 1:
            wait_scatter(ci - 1, (ci + 2) % _NBUF)
        if ci + 2 < nch:
            issue_tok(ci + 2, (ci + 2) % _NBUF)
        wait_tok(ci, s)
        issue_scatter(ci, s)

    issue_tok(0, 0)
    issue_tok(1, 1)
    step(0, 0)
    step(1, 1)

    def outer(g, _):
        for sp in range(_NBUF):
            ci = 2 + g * _NBUF + sp
            s = (2 + sp) % _NBUF
            wait_scatter(ci - 1, (s + 2) % _NBUF)
            issue_tok(ci + 2, (s + 2) % _NBUF)
            wait_tok(ci, s)
            issue_scatter(ci, s)
        return ()

    lax.fori_loop(0, (nch - 4) // _NBUF, outer, (), unroll=False)

    step(nch - 2, (nch - 2) % _NBUF)
    step(nch - 1, (nch - 1) % _NBUF)
    wait_scatter(nch - 1, (nch - 1) % _NBUF)


def _add_body(pos_ref, ptab_ref, tok_ref, out_ref, *, lp):
    ll = ptab_ref.shape[0]
    bb = out_ref.shape[0]
    posf = pos_ref[...]
    iota = lax.broadcasted_iota(jnp.int32, (1, ll), 1)
    onehot = (posf == iota).astype(jnp.float32)
    pe = lax.dot_general(onehot, ptab_ref[...], (((1,), (0,)), ((), ())),
                         preferred_element_type=jnp.float32)
    sm = (tok_ref[...] + pe).reshape(bb, lp, _D)
    out_ref[...] = sm[:, :ll, :]


def kernel(token_table, pos_table, tokens, positions):
    b, l = tokens.shape
    lp = (l + 7) // 8 * 8
    bt = b * l
    per_w = bt // _NW
    assert per_w % _C == 0 and (per_w // _C - 4) % _NBUF == 0

    tidx = tokens.reshape(bt).astype(jnp.int32)

    mesh = plsc.VectorSubcoreMesh(core_axis_name="c", subcore_axis_name="s")
    gather = pl.kernel(
        functools.partial(_gather_body, per_w=per_w, ll=l, lp=lp),
        mesh=mesh,
        compiler_params=pltpu.CompilerParams(needs_layout_passes=False),
        out_type=jax.ShapeDtypeStruct((b * lp, _D), jnp.float32),
        scratch_types=[
            pltpu.VMEM((per_w,), jnp.int32),
        ] + [pltpu.VMEM((_C, _D), jnp.float32)] * _NBUF
          + [pltpu.SemaphoreType.DMA] * (2 * _NBUF),
    )
    tok_rows = gather(token_table, tidx)

    pos_pad = jnp.pad(positions.astype(jnp.int32),
                      ((0, 0), (0, lp - l))).reshape(b * lp, 1)

    add = pl.pallas_call(
        functools.partial(_add_body, lp=lp),
        grid=(b // _BB,),
        in_specs=[
            pl.BlockSpec((_BB * lp, 1), lambda i: (i, 0)),
            pl.BlockSpec((l, _D), lambda i: (0, 0)),
            pl.BlockSpec((_BB * lp, _D), lambda i: (i, 0)),
        ],
        out_specs=pl.BlockSpec((_BB, l, _D), lambda i: (i, 0, 0)),
        out_shape=jax.ShapeDtypeStruct((b, l, _D), jnp.float32),
    )
    return add(pos_pad, pos_table, tok_rows)


# speedup vs baseline: 1.1257x; 1.0014x over previous
import functools

import jax
import jax.numpy as jnp
from jax import lax
from jax.experimental import pallas as pl
from jax.experimental.pallas import tpu as pltpu
from jax.experimental.pallas import tpu_sc as plsc

_D = 768
_LANES = 16
_NC = 2
_NS = 16
_NW = _NC * _NS
_C = 32
_NBUF = 4
_BB = 32


def _gather_body(tok_hbm, tidx_hbm, out_hbm, tidx_v,
                 t0, t1, t2, t3, g0, g1, g2, g3, s0, s1, s2, s3,
                 *, per_w, ll, lp):
    wid = lax.axis_index("s") * _NC + lax.axis_index("c")
    base = wid * per_w
    nch = per_w // _C
    lane = lax.iota(jnp.int32, _LANES)
    tbuf = (t0, t1, t2, t3)
    gsem = (g0, g1, g2, g3)
    ssem = (s0, s1, s2, s3)

    pltpu.sync_copy(tidx_hbm.at[pl.ds(base, per_w)], tidx_v)

    def issue_tok(ci, s):
        pltpu.async_copy(tok_hbm.at[tidx_v.at[pl.ds(ci * _C, _C)]],
                         tbuf[s], gsem[s])

    def wait_tok(ci, s):
        pltpu.make_async_copy(tok_hbm.at[tidx_v.at[pl.ds(ci * _C, _C)]],
                              tbuf[s], gsem[s]).wait()

    def out_rows(ci, k):
        r = base + ci * _C + k * _LANES + lane
        bv = r // ll
        return bv * lp + (r - bv * ll)

    def issue_scatter(ci, s):
        for k in range(_C // _LANES):
            pltpu.async_copy(tbuf[s].at[pl.ds(k * _LANES, _LANES)],
                             out_hbm.at[out_rows(ci, k)], ssem[s])

    def wait_scatter(ci, s):
        for k in range(_C // _LANES):
            pltpu.make_async_copy(tbuf[s].at[pl.ds(k * _LANES, _LANES)],
                                  out_hbm.at[out_rows(ci, k)],
                                  ssem[s]).wait()

    def step(ci, s):
        if ci >= 2:
            wait_scatter(ci - 2, (ci + 2) % _NBUF)
        if ci + 2 < nch:
            issue_tok(ci + 2, (ci + 2) % _NBUF)
        wait_tok(ci, s)
        issue_scatter(ci, s)

    issue_tok(0, 0)
    issue_tok(1, 1)
    step(0, 0)
    step(1, 1)

    def outer(g, _):
        for sp in range(_NBUF):
            ci = 2 + g * _NBUF + sp
            s = (2 + sp) % _NBUF
            wait_scatter(ci - 2, sp)
            issue_tok(ci + 2, sp)
            wait_tok(ci, (2 + sp) % _NBUF)
            issue_scatter(ci, (2 + sp) % _NBUF)
        return ()

    lax.fori_loop(0, (nch - 4) // _NBUF, outer, (), unroll=False)

    step(nch - 2, (nch - 2) % _NBUF)
    step(nch - 1, (nch - 1) % _NBUF)
    wait_scatter(nch - 2, (nch - 2) % _NBUF)
    wait_scatter(nch - 1, (nch - 1) % _NBUF)


def _add_body(pos_ref, ptab_ref, tok_ref, out_ref, *, lp):
    ll = ptab_ref.shape[0]
    bb = out_ref.shape[0]
    posf = pos_ref[...]
    iota = lax.broadcasted_iota(jnp.int32, (1, ll), 1)
    onehot = (posf == iota).astype(jnp.float32)
    pe = lax.dot_general(onehot, ptab_ref[...], (((1,), (0,)), ((), ())),
                         preferred_element_type=jnp.float32)
    sm = (tok_ref[...] + pe).reshape(bb, lp, _D)
    out_ref[...] = sm[:, :ll, :]


def kernel(token_table, pos_table, tokens, positions):
    b, l = tokens.shape
    lp = (l + 7) // 8 * 8
    bt = b * l
    per_w = bt // _NW
    assert per_w % _C == 0 and (per_w // _C - 4) % _NBUF == 0

    tidx = tokens.reshape(bt).astype(jnp.int32)

    mesh = plsc.VectorSubcoreMesh(core_axis_name="c", subcore_axis_name="s")
    gather = pl.kernel(
        functools.partial(_gather_body, per_w=per_w, ll=l, lp=lp),
        mesh=mesh,
        compiler_params=pltpu.CompilerParams(needs_layout_passes=False),
        out_type=jax.ShapeDtypeStruct((b * lp, _D), jnp.float32),
        scratch_types=[
            pltpu.VMEM((per_w,), jnp.int32),
        ] + [pltpu.VMEM((_C, _D), jnp.float32)] * _NBUF
          + [pltpu.SemaphoreType.DMA] * (2 * _NBUF),
    )
    tok_rows = gather(token_table, tidx)

    pos_pad = jnp.pad(positions.astype(jnp.int32),
                      ((0, 0), (0, lp - l))).reshape(b * lp, 1)

    add = pl.pallas_call(
        functools.partial(_add_body, lp=lp),
        grid=(b // _BB,),
        in_specs=[
            pl.BlockSpec((_BB * lp, 1), lambda i: (i, 0)),
            pl.BlockSpec((l, _D), lambda i: (0, 0)),
            pl.BlockSpec((_BB * lp, _D), lambda i: (i, 0)),
        ],
        out_specs=pl.BlockSpec((_BB, l, _D), lambda i: (i, 0, 0)),
        out_shape=jax.ShapeDtypeStruct((b, l, _D), jnp.float32),
    )
    return add(pos_pad, pos_table, tok_rows)
